# Initial kernel scaffold; baseline (speedup 1.0000x reference)
import jax
import jax.numpy as jnp
from jax.experimental import pallas as pl


def kernel(x, edge_index, bn_gamma, bn_beta, W1l, b1, W1r, W2l, b2, W2r):
    raise NotImplementedError("write your pallas kernel here")


# speedup vs baseline: 2.5657x; 2.5657x over previous
import functools

import jax
import jax.numpy as jnp
from jax import lax
from jax.experimental import pallas as pl
from jax.experimental.pallas import tpu as pltpu
from jax.experimental.pallas import tpu_sc as plsc

N = 10000
E = 320000
D = 128
EPS = 1e-5

NC = 2
NS = 16
NW = NC * NS
EPW = E // NW
EPW_PAD = 10240
E_PAD = NW * EPW_PAD
CHUNK = 32
NCHUNK = EPW_PAD // CHUNK
NSTAGE = 10
SCH = NCHUNK // NSTAGE
SPAIR = SCH // 2
ROWS_PER_TILE = 640
NPAD = NS * ROWS_PER_TILE
ZBLK = 16
CNTW = 16

_f32 = jnp.float32


def _make_sc_aggregate():
  mesh = plsc.VectorSubcoreMesh(core_axis_name="c", subcore_axis_name="s")

  scratch = [
      pltpu.VMEM((SCH, CHUNK), jnp.int32),
      pltpu.VMEM((SCH, CHUNK), jnp.int32),
      pltpu.VMEM((CHUNK, D), _f32),
      pltpu.VMEM((CHUNK, D), _f32),
      pltpu.VMEM((ZBLK, D), _f32),
      pltpu.VMEM((ZBLK,), jnp.int32),
      pltpu.VMEM_SHARED((NPAD, D), _f32),
      pltpu.SemaphoreType.DMA,
      pltpu.SemaphoreType.DMA,
  ]

  def body(h_hbm, src_hbm, dst_hbm, agg_out,
           src_idx, dst_idx, rows0, rows1, zrows, zidx, acc_sh, sem0, sem1):
    cid = lax.axis_index("c")
    sid = lax.axis_index("s")
    wid = cid * NS + sid
    r0 = sid * ROWS_PER_TILE

    z16 = jnp.zeros((16,), _f32)
    iota16 = lax.iota(jnp.int32, 16)

    def zfill_body(r, carry):
      for j in range(D // 16):
        zrows[r, pl.ds(16 * j, 16)] = z16
      return carry

    lax.fori_loop(0, ZBLK, zfill_body, 0)

    def zero_blk(k, carry):
      zidx[...] = iota16 + (r0 + ZBLK * k)
      pltpu.sync_copy(zrows, acc_sh.at[zidx])
      return carry

    lax.fori_loop(0, ROWS_PER_TILE // ZBLK, zero_blk, 0)

    plsc.subcore_barrier()

    for st in range(NSTAGE):
      pltpu.sync_copy(src_hbm.at[wid * NSTAGE + st], src_idx)
      pltpu.sync_copy(dst_hbm.at[wid * NSTAGE + st], dst_idx)

      pltpu.make_async_copy(h_hbm.at[src_idx.at[0]], rows0, sem0).start()

      def pair_body(c2, carry, prefetch):
        c = 2 * c2
        pltpu.make_async_copy(h_hbm.at[src_idx.at[c]], rows0, sem0).wait()
        pltpu.make_async_copy(h_hbm.at[src_idx.at[c + 1]], rows1, sem1).start()
        pltpu.sync_copy(rows0, acc_sh.at[dst_idx.at[c]], add=True)
        pltpu.make_async_copy(h_hbm.at[src_idx.at[c + 1]], rows1, sem1).wait()
        if prefetch:
          pltpu.make_async_copy(h_hbm.at[src_idx.at[c + 2]], rows0, sem0).start()
        pltpu.sync_copy(rows1, acc_sh.at[dst_idx.at[c + 1]], add=True)
        return carry

      lax.fori_loop(0, SPAIR - 1,
                    functools.partial(pair_body, prefetch=True), 0)
      pair_body(SPAIR - 1, 0, prefetch=False)

    plsc.subcore_barrier()

    def out_blk(k, carry):
      zidx[...] = iota16 + (r0 + ZBLK * k)
      pltpu.sync_copy(acc_sh.at[zidx], zrows)
      pltpu.sync_copy(zrows,
                      agg_out.at[pl.ds(cid * NPAD + r0 + ZBLK * k, ZBLK)])
      return carry

    lax.fori_loop(0, ROWS_PER_TILE // ZBLK, out_blk, 0)

  return pl.kernel(body,
                   out_type=jax.ShapeDtypeStruct((NC * NPAD, D), _f32),
                   mesh=mesh, scratch_types=scratch)


def _make_sc_count():
  mesh = plsc.VectorSubcoreMesh(core_axis_name="c", subcore_axis_name="s")

  scratch = [
      pltpu.VMEM((SCH, CHUNK), jnp.int32),
      pltpu.VMEM((CHUNK, D), _f32),
      pltpu.VMEM((ZBLK, D), _f32),
      pltpu.VMEM((ZBLK,), jnp.int32),
      pltpu.VMEM_SHARED((NPAD, D), _f32),
  ]

  def body(dst_hbm, cnt_out, dst_idx, ones_rows, zrows, zidx, acc_sh):
    cid = lax.axis_index("c")
    sid = lax.axis_index("s")
    wid = cid * NS + sid
    r0 = sid * ROWS_PER_TILE

    z16 = jnp.zeros((16,), _f32)
    iota16 = lax.iota(jnp.int32, 16)

    def fill_body(r, carry):
      for j in range(D // 16):
        zrows[r % ZBLK, pl.ds(16 * j, 16)] = z16
        ones_rows[r, pl.ds(16 * j, 16)] = z16 + 1.0
      return carry

    lax.fori_loop(0, CHUNK, fill_body, 0)

    def zero_blk(k, carry):
      zidx[...] = iota16 + (r0 + ZBLK * k)
      pltpu.sync_copy(zrows, acc_sh.at[zidx])
      return carry

    lax.fori_loop(0, ROWS_PER_TILE // ZBLK, zero_blk, 0)

    plsc.subcore_barrier()

    for st in range(NSTAGE):
      pltpu.sync_copy(dst_hbm.at[wid * NSTAGE + st], dst_idx)

      def chunk_body(c, carry):
        pltpu.sync_copy(ones_rows, acc_sh.at[dst_idx.at[c]], add=True)
        return carry

      lax.fori_loop(0, SCH, chunk_body, 0)

    plsc.subcore_barrier()

    def out_blk(k, carry):
      zidx[...] = iota16 + (r0 + ZBLK * k)
      pltpu.sync_copy(acc_sh.at[zidx], zrows)
      pltpu.sync_copy(zrows,
                      cnt_out.at[pl.ds(cid * NPAD + r0 + ZBLK * k, ZBLK)])
      return carry

    lax.fori_loop(0, ROWS_PER_TILE // ZBLK, out_blk, 0)

  return pl.kernel(body,
                   out_type=jax.ShapeDtypeStruct((NC * NPAD, D), _f32),
                   mesh=mesh, scratch_types=scratch)


def _bn_body(x_ref, g_ref, b_ref, out_ref):
  x = x_ref[...]
  mu = jnp.mean(x, axis=0, keepdims=True)
  d = x - mu
  var = jnp.mean(d * d, axis=0, keepdims=True)
  out_ref[...] = d * lax.rsqrt(var + EPS) * g_ref[...] + b_ref[...]


def _batch_norm(x, gamma, beta):
  return pl.pallas_call(
      _bn_body,
      out_shape=jax.ShapeDtypeStruct((N, D), _f32),
  )(x, gamma.reshape(1, D), beta.reshape(1, D))


_ROWBLK = 2048


def _make_combine(relu: bool, residual: bool):
  def body(*refs):
    if residual:
      agg_ref, cnt_ref, h_ref, wl_ref, wr_ref, b_ref, x_ref, out_ref = refs
    else:
      agg_ref, cnt_ref, h_ref, wl_ref, wr_ref, b_ref, out_ref = refs
    cnt = cnt_ref[0, :, 0:1] + cnt_ref[1, :, 0:1]
    mean = (agg_ref[0] + agg_ref[1]) / jnp.maximum(cnt, 1.0)
    dn = (((1,), (1,)), ((), ()))
    acc = lax.dot_general(mean, wl_ref[...], dn,
                          precision=lax.Precision.HIGHEST,
                          preferred_element_type=_f32)
    acc = acc + lax.dot_general(h_ref[...], wr_ref[...], dn,
                                precision=lax.Precision.HIGHEST,
                                preferred_element_type=_f32)
    acc = acc + b_ref[...]
    if residual:
      acc = acc + x_ref[...]
    if relu:
      acc = jnp.maximum(acc, 0.0)
    out_ref[...] = acc

  grid = (NPAD // _ROWBLK,)
  in_specs = [
      pl.BlockSpec((NC, _ROWBLK, D), lambda i: (0, i, 0)),
      pl.BlockSpec((NC, _ROWBLK, D), lambda i: (0, i, 0)),
      pl.BlockSpec((_ROWBLK, D), lambda i: (i, 0)),
      pl.BlockSpec((D, D), lambda i: (0, 0)),
      pl.BlockSpec((D, D), lambda i: (0, 0)),
      pl.BlockSpec((1, D), lambda i: (0, 0)),
  ]
  if residual:
    in_specs.append(pl.BlockSpec((_ROWBLK, D), lambda i: (i, 0)))

  return pl.pallas_call(
      body,
      grid=grid,
      in_specs=in_specs,
      out_specs=pl.BlockSpec((_ROWBLK, D), lambda i: (i, 0)),
      out_shape=jax.ShapeDtypeStruct((N, D), _f32),
  )


_sc_agg = _make_sc_aggregate()
_sc_count = _make_sc_count()
_combine_relu = _make_combine(relu=True, residual=False)
_combine_res = _make_combine(relu=False, residual=True)


def kernel(x, edge_index, bn_gamma, bn_beta, W1l, b1, W1r, W2l, b2, W2r):
  pad = E_PAD - E
  src = jnp.concatenate([edge_index[0], jnp.zeros((pad,), jnp.int32)])
  dst = jnp.concatenate([edge_index[1], jnp.full((pad,), N, jnp.int32)])
  src = src.reshape(NW * NSTAGE, SCH, CHUNK)
  dst = dst.reshape(NW * NSTAGE, SCH, CHUNK)

  h = _batch_norm(x, bn_gamma, bn_beta)
  cnt = _sc_count(dst).reshape(NC, NPAD, D)
  agg1 = _sc_agg(h, src, dst).reshape(NC, NPAD, D)
  h1 = _combine_relu(agg1, cnt, h, W1l, W1r, b1.reshape(1, D))
  agg2 = _sc_agg(h1, src, dst).reshape(NC, NPAD, D)
  out = _combine_res(agg2, cnt, h1, W2l, W2r, b2.reshape(1, D), x)
  return out


# speedup vs baseline: 2.9326x; 1.1430x over previous
import functools

import jax
import jax.numpy as jnp
from jax import lax
from jax.experimental import pallas as pl
from jax.experimental.pallas import tpu as pltpu
from jax.experimental.pallas import tpu_sc as plsc

N = 10000
E = 320000
D = 128
EPS = 1e-5

NC = 2
NS = 16
NW = NC * NS
EPW = E // NW
EPW_PAD = 10240
E_PAD = NW * EPW_PAD
CHUNK = 64
NCHUNK = EPW_PAD // CHUNK
NSTAGE = 10
SCH = NCHUNK // NSTAGE
SPAIR = SCH // 2
ROWS_PER_TILE = 640
NPAD = NS * ROWS_PER_TILE
ZBLK = 16
CNTW = 16

_f32 = jnp.float32


def _make_sc_aggregate():
  mesh = plsc.VectorSubcoreMesh(core_axis_name="c", subcore_axis_name="s")

  scratch = [
      pltpu.VMEM((SCH, CHUNK), jnp.int32),
      pltpu.VMEM((SCH, CHUNK), jnp.int32),
      pltpu.VMEM((CHUNK, D), _f32),
      pltpu.VMEM((CHUNK, D), _f32),
      pltpu.VMEM((ZBLK, D), _f32),
      pltpu.VMEM((ZBLK,), jnp.int32),
      pltpu.VMEM_SHARED((NPAD, D), _f32),
      pltpu.SemaphoreType.DMA,
      pltpu.SemaphoreType.DMA,
  ]

  def body(h_hbm, src_hbm, dst_hbm, agg_out,
           src_idx, dst_idx, rows0, rows1, zrows, zidx, acc_sh, sem0, sem1):
    cid = lax.axis_index("c")
    sid = lax.axis_index("s")
    wid = cid * NS + sid
    r0 = sid * ROWS_PER_TILE

    z16 = jnp.zeros((16,), _f32)
    iota16 = lax.iota(jnp.int32, 16)

    def zfill_body(r, carry):
      for j in range(D // 16):
        zrows[r, pl.ds(16 * j, 16)] = z16
      return carry

    lax.fori_loop(0, ZBLK, zfill_body, 0)

    def zero_blk(k, carry):
      zidx[...] = iota16 + (r0 + ZBLK * k)
      pltpu.sync_copy(zrows, acc_sh.at[zidx])
      return carry

    lax.fori_loop(0, ROWS_PER_TILE // ZBLK, zero_blk, 0)

    plsc.subcore_barrier()

    for st in range(NSTAGE):
      pltpu.sync_copy(src_hbm.at[wid * NSTAGE + st], src_idx)
      pltpu.sync_copy(dst_hbm.at[wid * NSTAGE + st], dst_idx)

      pltpu.make_async_copy(h_hbm.at[src_idx.at[0]], rows0, sem0).start()

      def pair_body(c2, carry, prefetch):
        c = 2 * c2
        pltpu.make_async_copy(h_hbm.at[src_idx.at[c]], rows0, sem0).wait()
        pltpu.make_async_copy(h_hbm.at[src_idx.at[c + 1]], rows1, sem1).start()
        pltpu.sync_copy(rows0, acc_sh.at[dst_idx.at[c]], add=True)
        pltpu.make_async_copy(h_hbm.at[src_idx.at[c + 1]], rows1, sem1).wait()
        if prefetch:
          pltpu.make_async_copy(h_hbm.at[src_idx.at[c + 2]], rows0, sem0).start()
        pltpu.sync_copy(rows1, acc_sh.at[dst_idx.at[c + 1]], add=True)
        return carry

      lax.fori_loop(0, SPAIR - 1,
                    functools.partial(pair_body, prefetch=True), 0)
      pair_body(SPAIR - 1, 0, prefetch=False)

    plsc.subcore_barrier()

    def out_blk(k, carry):
      zidx[...] = iota16 + (r0 + ZBLK * k)
      pltpu.sync_copy(acc_sh.at[zidx], zrows)
      pltpu.sync_copy(zrows,
                      agg_out.at[pl.ds(cid * NPAD + r0 + ZBLK * k, ZBLK)])
      return carry

    lax.fori_loop(0, ROWS_PER_TILE // ZBLK, out_blk, 0)

  return pl.kernel(body,
                   out_type=jax.ShapeDtypeStruct((NC * NPAD, D), _f32),
                   mesh=mesh, scratch_types=scratch)


def _make_sc_count():
  mesh = plsc.VectorSubcoreMesh(core_axis_name="c", subcore_axis_name="s")

  scratch = [
      pltpu.VMEM((SCH, CHUNK), jnp.int32),
      pltpu.VMEM((CHUNK, D), _f32),
      pltpu.VMEM((ZBLK, D), _f32),
      pltpu.VMEM((ZBLK,), jnp.int32),
      pltpu.VMEM_SHARED((NPAD, D), _f32),
  ]

  def body(dst_hbm, cnt_out, dst_idx, ones_rows, zrows, zidx, acc_sh):
    cid = lax.axis_index("c")
    sid = lax.axis_index("s")
    wid = cid * NS + sid
    r0 = sid * ROWS_PER_TILE

    z16 = jnp.zeros((16,), _f32)
    iota16 = lax.iota(jnp.int32, 16)

    def fill_body(r, carry):
      for j in range(D // 16):
        zrows[r % ZBLK, pl.ds(16 * j, 16)] = z16
        ones_rows[r, pl.ds(16 * j, 16)] = z16 + 1.0
      return carry

    lax.fori_loop(0, CHUNK, fill_body, 0)

    def zero_blk(k, carry):
      zidx[...] = iota16 + (r0 + ZBLK * k)
      pltpu.sync_copy(zrows, acc_sh.at[zidx])
      return carry

    lax.fori_loop(0, ROWS_PER_TILE // ZBLK, zero_blk, 0)

    plsc.subcore_barrier()

    for st in range(NSTAGE):
      pltpu.sync_copy(dst_hbm.at[wid * NSTAGE + st], dst_idx)

      def chunk_body(c, carry):
        pltpu.sync_copy(ones_rows, acc_sh.at[dst_idx.at[c]], add=True)
        return carry

      lax.fori_loop(0, SCH, chunk_body, 0)

    plsc.subcore_barrier()

    def out_blk(k, carry):
      zidx[...] = iota16 + (r0 + ZBLK * k)
      pltpu.sync_copy(acc_sh.at[zidx], zrows)
      pltpu.sync_copy(zrows,
                      cnt_out.at[pl.ds(cid * NPAD + r0 + ZBLK * k, ZBLK)])
      return carry

    lax.fori_loop(0, ROWS_PER_TILE // ZBLK, out_blk, 0)

  return pl.kernel(body,
                   out_type=jax.ShapeDtypeStruct((NC * NPAD, D), _f32),
                   mesh=mesh, scratch_types=scratch)


def _bn_body(x_ref, g_ref, b_ref, out_ref):
  x = x_ref[...]
  mu = jnp.mean(x, axis=0, keepdims=True)
  d = x - mu
  var = jnp.mean(d * d, axis=0, keepdims=True)
  out_ref[...] = d * lax.rsqrt(var + EPS) * g_ref[...] + b_ref[...]


def _batch_norm(x, gamma, beta):
  return pl.pallas_call(
      _bn_body,
      out_shape=jax.ShapeDtypeStruct((N, D), _f32),
  )(x, gamma.reshape(1, D), beta.reshape(1, D))


_ROWBLK = 2048


def _make_combine(relu: bool, residual: bool):
  def body(*refs):
    if residual:
      agg_ref, cnt_ref, h_ref, wl_ref, wr_ref, b_ref, x_ref, out_ref = refs
    else:
      agg_ref, cnt_ref, h_ref, wl_ref, wr_ref, b_ref, out_ref = refs
    cnt = cnt_ref[0, :, 0:1] + cnt_ref[1, :, 0:1]
    mean = (agg_ref[0] + agg_ref[1]) / jnp.maximum(cnt, 1.0)
    dn = (((1,), (1,)), ((), ()))
    acc = lax.dot_general(mean, wl_ref[...], dn,
                          precision=lax.Precision.HIGHEST,
                          preferred_element_type=_f32)
    acc = acc + lax.dot_general(h_ref[...], wr_ref[...], dn,
                                precision=lax.Precision.HIGHEST,
                                preferred_element_type=_f32)
    acc = acc + b_ref[...]
    if residual:
      acc = acc + x_ref[...]
    if relu:
      acc = jnp.maximum(acc, 0.0)
    out_ref[...] = acc

  grid = (NPAD // _ROWBLK,)
  in_specs = [
      pl.BlockSpec((NC, _ROWBLK, D), lambda i: (0, i, 0)),
      pl.BlockSpec((NC, _ROWBLK, D), lambda i: (0, i, 0)),
      pl.BlockSpec((_ROWBLK, D), lambda i: (i, 0)),
      pl.BlockSpec((D, D), lambda i: (0, 0)),
      pl.BlockSpec((D, D), lambda i: (0, 0)),
      pl.BlockSpec((1, D), lambda i: (0, 0)),
  ]
  if residual:
    in_specs.append(pl.BlockSpec((_ROWBLK, D), lambda i: (i, 0)))

  return pl.pallas_call(
      body,
      grid=grid,
      in_specs=in_specs,
      out_specs=pl.BlockSpec((_ROWBLK, D), lambda i: (i, 0)),
      out_shape=jax.ShapeDtypeStruct((N, D), _f32),
  )


_sc_agg = _make_sc_aggregate()
_sc_count = _make_sc_count()
_combine_relu = _make_combine(relu=True, residual=False)
_combine_res = _make_combine(relu=False, residual=True)


def kernel(x, edge_index, bn_gamma, bn_beta, W1l, b1, W1r, W2l, b2, W2r):
  pad = E_PAD - E
  src = jnp.concatenate([edge_index[0], jnp.zeros((pad,), jnp.int32)])
  dst = jnp.concatenate([edge_index[1], jnp.full((pad,), N, jnp.int32)])
  src = src.reshape(NW * NSTAGE, SCH, CHUNK)
  dst = dst.reshape(NW * NSTAGE, SCH, CHUNK)

  h = _batch_norm(x, bn_gamma, bn_beta)
  cnt = _sc_count(dst).reshape(NC, NPAD, D)
  agg1 = _sc_agg(h, src, dst).reshape(NC, NPAD, D)
  h1 = _combine_relu(agg1, cnt, h, W1l, W1r, b1.reshape(1, D))
  agg2 = _sc_agg(h1, src, dst).reshape(NC, NPAD, D)
  out = _combine_res(agg2, cnt, h1, W2l, W2r, b2.reshape(1, D), x)
  return out


# speedup vs baseline: 2.9769x; 1.0151x over previous
import functools

import jax
import jax.numpy as jnp
from jax import lax
from jax.experimental import pallas as pl
from jax.experimental.pallas import tpu as pltpu
from jax.experimental.pallas import tpu_sc as plsc

N = 10000
E = 320000
D = 128
EPS = 1e-5

NC = 2
NS = 16
NW = NC * NS
EPW = E // NW
EPW_PAD = 10240
E_PAD = NW * EPW_PAD
CHUNK = 64
NCHUNK = EPW_PAD // CHUNK
NSTAGE = 10
SCH = NCHUNK // NSTAGE
SPAIR = SCH // 2
ROWS_PER_TILE = 640
NPAD = NS * ROWS_PER_TILE
ZBLK = 16
CNTW = 16

_f32 = jnp.float32


def _make_sc_aggregate():
  mesh = plsc.VectorSubcoreMesh(core_axis_name="c", subcore_axis_name="s")

  scratch = [
      pltpu.VMEM((SCH, CHUNK), jnp.int32),
      pltpu.VMEM((SCH, CHUNK), jnp.int32),
      pltpu.VMEM((CHUNK, D), _f32),
      pltpu.VMEM((CHUNK, D), _f32),
      pltpu.VMEM((ZBLK, D), _f32),
      pltpu.VMEM((ZBLK,), jnp.int32),
      pltpu.VMEM_SHARED((NPAD, D), _f32),
      pltpu.SemaphoreType.DMA,
      pltpu.SemaphoreType.DMA,
      pltpu.SemaphoreType.DMA,
      pltpu.SemaphoreType.DMA,
  ]

  def body(h_hbm, src_hbm, dst_hbm, agg_out,
           src_idx, dst_idx, rows0, rows1, zrows, zidx, acc_sh,
           sem0, sem1, sem_s0, sem_s1):
    cid = lax.axis_index("c")
    sid = lax.axis_index("s")
    wid = cid * NS + sid
    r0 = sid * ROWS_PER_TILE

    z16 = jnp.zeros((16,), _f32)
    iota16 = lax.iota(jnp.int32, 16)

    def zfill_body(r, carry):
      for j in range(D // 16):
        zrows[r, pl.ds(16 * j, 16)] = z16
      return carry

    lax.fori_loop(0, ZBLK, zfill_body, 0)

    def zero_blk(k, carry):
      zidx[...] = iota16 + (r0 + ZBLK * k)
      pltpu.sync_copy(zrows, acc_sh.at[zidx])
      return carry

    lax.fori_loop(0, ROWS_PER_TILE // ZBLK, zero_blk, 0)

    plsc.subcore_barrier()

    for st in range(NSTAGE):
      pltpu.sync_copy(src_hbm.at[wid * NSTAGE + st], src_idx)
      pltpu.sync_copy(dst_hbm.at[wid * NSTAGE + st], dst_idx)

      pltpu.make_async_copy(h_hbm.at[src_idx.at[0]], rows0, sem0).start()
      pltpu.make_async_copy(h_hbm.at[src_idx.at[1]], rows1, sem1).start()

      def pair_body(c2, carry, prefetch):
        c = 2 * c2
        pltpu.make_async_copy(h_hbm.at[src_idx.at[c]], rows0, sem0).wait()
        s0 = pltpu.async_copy(rows0, acc_sh.at[dst_idx.at[c]], sem_s0,
                              add=True)
        pltpu.make_async_copy(h_hbm.at[src_idx.at[c + 1]], rows1, sem1).wait()
        s1 = pltpu.async_copy(rows1, acc_sh.at[dst_idx.at[c + 1]], sem_s1,
                              add=True)
        s0.wait()
        if prefetch:
          pltpu.make_async_copy(h_hbm.at[src_idx.at[c + 2]], rows0, sem0).start()
        s1.wait()
        if prefetch:
          pltpu.make_async_copy(h_hbm.at[src_idx.at[c + 3]], rows1, sem1).start()
        return carry

      lax.fori_loop(0, SPAIR - 1,
                    functools.partial(pair_body, prefetch=True), 0)
      pair_body(SPAIR - 1, 0, prefetch=False)

    plsc.subcore_barrier()

    def out_blk(k, carry):
      zidx[...] = iota16 + (r0 + ZBLK * k)
      pltpu.sync_copy(acc_sh.at[zidx], zrows)
      pltpu.sync_copy(zrows,
                      agg_out.at[pl.ds(cid * NPAD + r0 + ZBLK * k, ZBLK)])
      return carry

    lax.fori_loop(0, ROWS_PER_TILE // ZBLK, out_blk, 0)

  return pl.kernel(body,
                   out_type=jax.ShapeDtypeStruct((NC * NPAD, D), _f32),
                   mesh=mesh, scratch_types=scratch)


def _make_sc_count():
  mesh = plsc.VectorSubcoreMesh(core_axis_name="c", subcore_axis_name="s")

  scratch = [
      pltpu.VMEM((SCH, CHUNK), jnp.int32),
      pltpu.VMEM((CHUNK, D), _f32),
      pltpu.VMEM((ZBLK, D), _f32),
      pltpu.VMEM((ZBLK,), jnp.int32),
      pltpu.VMEM_SHARED((NPAD, D), _f32),
  ]

  def body(dst_hbm, cnt_out, dst_idx, ones_rows, zrows, zidx, acc_sh):
    cid = lax.axis_index("c")
    sid = lax.axis_index("s")
    wid = cid * NS + sid
    r0 = sid * ROWS_PER_TILE

    z16 = jnp.zeros((16,), _f32)
    iota16 = lax.iota(jnp.int32, 16)

    def fill_body(r, carry):
      for j in range(D // 16):
        zrows[r % ZBLK, pl.ds(16 * j, 16)] = z16
        ones_rows[r, pl.ds(16 * j, 16)] = z16 + 1.0
      return carry

    lax.fori_loop(0, CHUNK, fill_body, 0)

    def zero_blk(k, carry):
      zidx[...] = iota16 + (r0 + ZBLK * k)
      pltpu.sync_copy(zrows, acc_sh.at[zidx])
      return carry

    lax.fori_loop(0, ROWS_PER_TILE // ZBLK, zero_blk, 0)

    plsc.subcore_barrier()

    for st in range(NSTAGE):
      pltpu.sync_copy(dst_hbm.at[wid * NSTAGE + st], dst_idx)

      def chunk_body(c, carry):
        pltpu.sync_copy(ones_rows, acc_sh.at[dst_idx.at[c]], add=True)
        return carry

      lax.fori_loop(0, SCH, chunk_body, 0)

    plsc.subcore_barrier()

    def out_blk(k, carry):
      zidx[...] = iota16 + (r0 + ZBLK * k)
      pltpu.sync_copy(acc_sh.at[zidx], zrows)
      pltpu.sync_copy(zrows,
                      cnt_out.at[pl.ds(cid * NPAD + r0 + ZBLK * k, ZBLK)])
      return carry

    lax.fori_loop(0, ROWS_PER_TILE // ZBLK, out_blk, 0)

  return pl.kernel(body,
                   out_type=jax.ShapeDtypeStruct((NC * NPAD, D), _f32),
                   mesh=mesh, scratch_types=scratch)


def _bn_body(x_ref, g_ref, b_ref, out_ref):
  x = x_ref[...]
  mu = jnp.mean(x, axis=0, keepdims=True)
  d = x - mu
  var = jnp.mean(d * d, axis=0, keepdims=True)
  out_ref[...] = d * lax.rsqrt(var + EPS) * g_ref[...] + b_ref[...]


def _batch_norm(x, gamma, beta):
  return pl.pallas_call(
      _bn_body,
      out_shape=jax.ShapeDtypeStruct((N, D), _f32),
  )(x, gamma.reshape(1, D), beta.reshape(1, D))


_ROWBLK = 2048


def _make_combine(relu: bool, residual: bool):
  def body(*refs):
    if residual:
      agg_ref, cnt_ref, h_ref, wl_ref, wr_ref, b_ref, x_ref, out_ref = refs
    else:
      agg_ref, cnt_ref, h_ref, wl_ref, wr_ref, b_ref, out_ref = refs
    cnt = cnt_ref[0, :, 0:1] + cnt_ref[1, :, 0:1]
    mean = (agg_ref[0] + agg_ref[1]) / jnp.maximum(cnt, 1.0)
    dn = (((1,), (1,)), ((), ()))
    acc = lax.dot_general(mean, wl_ref[...], dn,
                          precision=lax.Precision.HIGHEST,
                          preferred_element_type=_f32)
    acc = acc + lax.dot_general(h_ref[...], wr_ref[...], dn,
                                precision=lax.Precision.HIGHEST,
                                preferred_element_type=_f32)
    acc = acc + b_ref[...]
    if residual:
      acc = acc + x_ref[...]
    if relu:
      acc = jnp.maximum(acc, 0.0)
    out_ref[...] = acc

  grid = (NPAD // _ROWBLK,)
  in_specs = [
      pl.BlockSpec((NC, _ROWBLK, D), lambda i: (0, i, 0)),
      pl.BlockSpec((NC, _ROWBLK, D), lambda i: (0, i, 0)),
      pl.BlockSpec((_ROWBLK, D), lambda i: (i, 0)),
      pl.BlockSpec((D, D), lambda i: (0, 0)),
      pl.BlockSpec((D, D), lambda i: (0, 0)),
      pl.BlockSpec((1, D), lambda i: (0, 0)),
  ]
  if residual:
    in_specs.append(pl.BlockSpec((_ROWBLK, D), lambda i: (i, 0)))

  return pl.pallas_call(
      body,
      grid=grid,
      in_specs=in_specs,
      out_specs=pl.BlockSpec((_ROWBLK, D), lambda i: (i, 0)),
      out_shape=jax.ShapeDtypeStruct((N, D), _f32),
  )


_sc_agg = _make_sc_aggregate()
_sc_count = _make_sc_count()
_combine_relu = _make_combine(relu=True, residual=False)
_combine_res = _make_combine(relu=False, residual=True)


def kernel(x, edge_index, bn_gamma, bn_beta, W1l, b1, W1r, W2l, b2, W2r):
  pad = E_PAD - E
  src = jnp.concatenate([edge_index[0], jnp.zeros((pad,), jnp.int32)])
  dst = jnp.concatenate([edge_index[1], jnp.full((pad,), N, jnp.int32)])
  src = src.reshape(NW * NSTAGE, SCH, CHUNK)
  dst = dst.reshape(NW * NSTAGE, SCH, CHUNK)

  h = _batch_norm(x, bn_gamma, bn_beta)
  cnt = _sc_count(dst).reshape(NC, NPAD, D)
  agg1 = _sc_agg(h, src, dst).reshape(NC, NPAD, D)
  h1 = _combine_relu(agg1, cnt, h, W1l, W1r, b1.reshape(1, D))
  agg2 = _sc_agg(h1, src, dst).reshape(NC, NPAD, D)
  out = _combine_res(agg2, cnt, h1, W2l, W2r, b2.reshape(1, D), x)
  return out
